# Initial kernel scaffold; baseline (speedup 1.0000x reference)
import jax
import jax.numpy as jnp
from jax.experimental import pallas as pl


def kernel(vectors, node_feats, node_specie, senders, receivers, W_up, w_sh, W1, W2, W3, W_down, W_skip):
    raise NotImplementedError("write your pallas kernel here")


# speedup vs baseline: 2.5469x; 2.5469x over previous
import functools
import math

import jax
import jax.numpy as jnp
from jax import lax
from jax.experimental import pallas as pl
from jax.experimental.pallas import tpu as pltpu
from jax.experimental.pallas import tpu_sc as plsc

N = 10000
E = 320000
D = 128
EB = 2560
SCB = 128
NBLK = E // SCB
NSUB = 16
NPAD = 10240
ROWS_PER_TEC = NPAD // NSUB
INV_SQRT_AVG = 1.0 / math.sqrt(32.0)


def _silu(x):
    return x * jax.nn.sigmoid(x)


def _node_prep_body(nf_ref, sp_ref, wup_ref, wskip_ref, h_ref, sc_ref):
    nf = nf_ref[:]
    h_ref[:] = jnp.dot(nf, wup_ref[:], preferred_element_type=jnp.float32)
    sp = sp_ref[:]
    acc = jnp.zeros_like(nf)
    for k in range(5):
        masked = jnp.where(sp == k, nf, 0.0)
        acc = acc + jnp.dot(masked, wskip_ref[k], preferred_element_type=jnp.float32)
    sc_ref[:] = acc


def _node_prep(node_feats, node_specie2d, W_up, W_skip):
    return pl.pallas_call(
        _node_prep_body,
        out_shape=[
            jax.ShapeDtypeStruct((N, D), jnp.float32),
            jax.ShapeDtypeStruct((N, D), jnp.float32),
        ],
    )(node_feats, node_specie2d, W_up, W_skip)


_SH_COEF = [
    math.sqrt(3.0), math.sqrt(3.0), math.sqrt(3.0),
    math.sqrt(15.0), math.sqrt(15.0), math.sqrt(5.0) / 2.0,
    math.sqrt(15.0), math.sqrt(15.0) / 2.0,
    math.sqrt(35.0 / 8.0), math.sqrt(105.0), math.sqrt(21.0 / 8.0),
    math.sqrt(7.0) / 2.0, math.sqrt(21.0 / 8.0), math.sqrt(105.0) / 2.0,
    math.sqrt(35.0 / 8.0),
]


def _edge_mlp_body(vt_ref, wsh_ref, q1_ref, w2_ref, w3_ref, out_ref):
    x = vt_ref[0:1]
    y = vt_ref[1:2]
    z = vt_ref[2:3]
    x2 = x * x + y * y + z * z
    l = jnp.sqrt(jnp.where(x2 == 0.0, 1.0, x2))
    invl = 1.0 / l
    ns = (lax.broadcasted_iota(jnp.int32, (8, 1), 0) + 1).astype(jnp.float32)
    l2 = l * l
    l5 = l2 * l2 * l
    env = 1.0 - 21.0 * l5 + 35.0 * l5 * l - 15.0 * l5 * l2
    cutoff = jnp.where(l < 1.0, env, 0.0)
    radT = (jnp.sqrt(2.0) * jnp.sin(ns * (jnp.pi * l))) * (cutoff * invl)
    ux, uy, uz = x * invl, y * invl, z * invl
    comps = [
        ux, uy, uz,
        ux * uy, uy * uz, 3.0 * uz * uz - 1.0, ux * uz, ux * ux - uy * uy,
        uy * (3.0 * ux * ux - uy * uy), ux * uy * uz,
        uy * (5.0 * uz * uz - 1.0), uz * (5.0 * uz * uz - 3.0),
        ux * (5.0 * uz * uz - 1.0), uz * (ux * ux - uy * uy),
        ux * (ux * ux - 3.0 * uy * uy),
    ]
    s = jnp.zeros_like(x)
    for k in range(15):
        s = s + (wsh_ref[0:1, k:k + 1] * _SH_COEF[k]) * comps[k]
    p = jnp.concatenate([radT, s], axis=0)
    r = lax.dot_general(p, q1_ref[:], (((0,), (0,)), ((), ())),
                        preferred_element_type=jnp.float32)
    m = _silu(r[:, :64])
    scol = r[:, 64:65]
    m = _silu(jnp.dot(m, w2_ref[:], preferred_element_type=jnp.float32))
    m = jnp.dot(m, w3_ref[:], preferred_element_type=jnp.float32)
    m = m * INV_SQRT_AVG
    out_ref[0] = m[:, :D]
    out_ref[1] = m[:, D:] * scol


def _edge_mlp(vectors_t, w_sh2d, Q1, W2, W3):
    grid = (E // EB,)
    return pl.pallas_call(
        _edge_mlp_body,
        grid=grid,
        in_specs=[
            pl.BlockSpec((3, EB), lambda i: (0, i)),
            pl.BlockSpec((1, 15), lambda i: (0, 0)),
            pl.BlockSpec((9, 128), lambda i: (0, 0)),
            pl.BlockSpec((64, 64), lambda i: (0, 0)),
            pl.BlockSpec((64, 2 * D), lambda i: (0, 0)),
        ],
        out_specs=pl.BlockSpec((2, EB, D), lambda i: (0, i, 0)),
        out_shape=jax.ShapeDtypeStruct((2, E, D), jnp.float32),
    )(vectors_t, w_sh2d, Q1, W2, W3)


@functools.cache
def _make_sc_message_pass():
    mesh = plsc.VectorSubcoreMesh(core_axis_name="c", subcore_axis_name="s",
                                  num_cores=2, num_subcores=NSUB)
    return functools.partial(
        pl.kernel,
        mesh=mesh,
        out_type=jax.ShapeDtypeStruct((2, NPAD, D), jnp.float32),
        scratch_types=[
            pltpu.VMEM((SCB,), jnp.int32),
            pltpu.VMEM((SCB,), jnp.int32),
            pltpu.VMEM((SCB, D), jnp.float32),
            pltpu.VMEM((SCB, D), jnp.float32),
            pltpu.VMEM_SHARED((NPAD, D), jnp.float32),
            pltpu.SemaphoreType.DMA,
        ],
    )(_sc_message_pass_body)


def _sc_message_pass_body(h_hbm, snd_hbm, rcv_hbm, w_hbm, out_hbm,
                          sidx, ridx, rows, wbuf, aggsh, sem):
    c = lax.axis_index("c")
    t = lax.axis_index("s")

    def _zero_row(r, carry):
        for cc in range(D // 16):
            rows[r, pl.ds(cc * 16, 16)] = jnp.zeros((16,), jnp.float32)
        return carry

    lax.fori_loop(0, SCB, _zero_row, 0)
    base = t * ROWS_PER_TEC
    for j in range(ROWS_PER_TEC // SCB):
        pltpu.sync_copy(rows, aggsh.at[pl.ds(base + j * SCB, SCB)])
    plsc.subcore_barrier()

    def _block(i, carry):
        b = t + i * NSUB

        @pl.when(b < NBLK)
        def _():
            off = b * SCB
            pltpu.sync_copy(snd_hbm.at[pl.ds(off, SCB)], sidx)
            pltpu.sync_copy(rcv_hbm.at[pl.ds(off, SCB)], ridx)
            pltpu.async_copy(h_hbm.at[sidx], rows, sem).wait()
            pltpu.sync_copy(w_hbm.at[c, pl.ds(off, SCB)], wbuf)

            def _mul(r, carry2):
                for cc in range(D // 16):
                    sl = pl.ds(cc * 16, 16)
                    rows[r, sl] = rows[r, sl] * wbuf[r, sl]
                return carry2

            lax.fori_loop(0, SCB, _mul, 0)
            pltpu.sync_copy(rows, aggsh.at[ridx], add=True)

        return carry

    lax.fori_loop(0, (NBLK + NSUB - 1) // NSUB, _block, 0)
    plsc.subcore_barrier()
    pltpu.sync_copy(aggsh.at[pl.ds(base, ROWS_PER_TEC)],
                    out_hbm.at[c, pl.ds(base, ROWS_PER_TEC)])


def _final_body(agg_ref, wd_ref, sc_ref, out_ref):
    wd = wd_ref[:]
    z = (jnp.dot(agg_ref[0, :N], wd[:D], preferred_element_type=jnp.float32)
         + jnp.dot(agg_ref[1, :N], wd[D:], preferred_element_type=jnp.float32))
    out_ref[:] = _silu(z) + sc_ref[:]


def _final(agg, W_down, sc):
    return pl.pallas_call(
        _final_body,
        out_shape=jax.ShapeDtypeStruct((N, D), jnp.float32),
    )(agg, W_down, sc)


def kernel(vectors, node_feats, node_specie, senders, receivers,
           W_up, w_sh, W1, W2, W3, W_down, W_skip):
    h, sc = _node_prep(node_feats,
                       node_specie.reshape(N, 1).astype(jnp.int32),
                       W_up, W_skip)
    Q1 = jnp.zeros((9, 128), jnp.float32)
    Q1 = Q1.at[:8, :64].set(W1).at[8, 64].set(1.0)
    wpack = _edge_mlp(vectors.T, w_sh.reshape(1, 15), Q1, W2, W3)
    agg = _make_sc_message_pass()(h, senders.astype(jnp.int32),
                                  receivers.astype(jnp.int32), wpack)
    return _final(agg, W_down, sc)


# speedup vs baseline: 3.8489x; 1.5112x over previous
import functools
import math

import jax
import jax.numpy as jnp
from jax import lax
from jax.experimental import pallas as pl
from jax.experimental.pallas import tpu as pltpu
from jax.experimental.pallas import tpu_sc as plsc

N = 10000
E = 320000
D = 128
EB = 2560
SCB = 128
NBLK = E // SCB
NCORE = 2
NSUB = 16
NWORK = NCORE * NSUB
NPAD = 10240
ROWS_PER_TEC = NPAD // NSUB
INV_SQRT_AVG = 1.0 / math.sqrt(32.0)


def _silu(x):
    return x * jax.nn.sigmoid(x)


def _node_prep_body(nf_ref, sp_ref, wup_ref, wskip_ref, h_ref, sc_ref):
    nf = nf_ref[:]
    h_ref[:] = jnp.dot(nf, wup_ref[:], preferred_element_type=jnp.float32)
    sp = sp_ref[:]
    acc = jnp.zeros_like(nf)
    for k in range(5):
        masked = jnp.where(sp == k, nf, 0.0)
        acc = acc + jnp.dot(masked, wskip_ref[k], preferred_element_type=jnp.float32)
    sc_ref[:] = acc


def _node_prep(node_feats, node_specie2d, W_up, W_skip):
    return pl.pallas_call(
        _node_prep_body,
        out_shape=[
            jax.ShapeDtypeStruct((N, D), jnp.float32),
            jax.ShapeDtypeStruct((N, D), jnp.float32),
        ],
    )(node_feats, node_specie2d, W_up, W_skip)


@functools.cache
def _make_sc_gather():
    mesh = plsc.VectorSubcoreMesh(core_axis_name="c", subcore_axis_name="s",
                                  num_cores=NCORE, num_subcores=NSUB)
    return functools.partial(
        pl.kernel,
        mesh=mesh,
        out_type=jax.ShapeDtypeStruct((E, D), jnp.float32),
        scratch_types=[
            pltpu.VMEM((SCB,), jnp.int32),
            pltpu.VMEM((SCB, D), jnp.float32),
            pltpu.SemaphoreType.DMA,
        ],
    )(_sc_gather_body)


def _sc_gather_body(h_hbm, snd_hbm, out_hbm, sidx, rows, sem):
    c = lax.axis_index("c")
    t = lax.axis_index("s")
    w = t * NCORE + c

    def _block(i, carry):
        b = w + i * NWORK

        @pl.when(b < NBLK)
        def _():
            off = b * SCB
            pltpu.sync_copy(snd_hbm.at[pl.ds(off, SCB)], sidx)
            pltpu.async_copy(h_hbm.at[sidx], rows, sem).wait()
            pltpu.sync_copy(rows, out_hbm.at[pl.ds(off, SCB)])

        return carry

    lax.fori_loop(0, (NBLK + NWORK - 1) // NWORK, _block, 0)


_SH_COEF = [
    math.sqrt(3.0), math.sqrt(3.0), math.sqrt(3.0),
    math.sqrt(15.0), math.sqrt(15.0), math.sqrt(5.0) / 2.0,
    math.sqrt(15.0), math.sqrt(15.0) / 2.0,
    math.sqrt(35.0 / 8.0), math.sqrt(105.0), math.sqrt(21.0 / 8.0),
    math.sqrt(7.0) / 2.0, math.sqrt(21.0 / 8.0), math.sqrt(105.0) / 2.0,
    math.sqrt(35.0 / 8.0),
]


def _edge_tc_body(vt_ref, msgs_ref, wsh_ref, q1_ref, w2_ref, w3_ref, wd_ref,
                  y_ref):
    x = vt_ref[0:1]
    y = vt_ref[1:2]
    z = vt_ref[2:3]
    x2 = x * x + y * y + z * z
    l = jnp.sqrt(jnp.where(x2 == 0.0, 1.0, x2))
    invl = 1.0 / l
    ns = (lax.broadcasted_iota(jnp.int32, (8, 1), 0) + 1).astype(jnp.float32)
    l2 = l * l
    l5 = l2 * l2 * l
    env = 1.0 - 21.0 * l5 + 35.0 * l5 * l - 15.0 * l5 * l2
    cutoff = jnp.where(l < 1.0, env, 0.0)
    radT = (jnp.sqrt(2.0) * jnp.sin(ns * (jnp.pi * l))) * (cutoff * invl)
    ux, uy, uz = x * invl, y * invl, z * invl
    comps = [
        ux, uy, uz,
        ux * uy, uy * uz, 3.0 * uz * uz - 1.0, ux * uz, ux * ux - uy * uy,
        uy * (3.0 * ux * ux - uy * uy), ux * uy * uz,
        uy * (5.0 * uz * uz - 1.0), uz * (5.0 * uz * uz - 3.0),
        ux * (5.0 * uz * uz - 1.0), uz * (ux * ux - uy * uy),
        ux * (ux * ux - 3.0 * uy * uy),
    ]
    s = jnp.zeros_like(x)
    for k in range(15):
        s = s + (wsh_ref[0:1, k:k + 1] * _SH_COEF[k]) * comps[k]
    p = jnp.concatenate([radT, s], axis=0)
    r = lax.dot_general(p, q1_ref[:], (((0,), (0,)), ((), ())),
                        preferred_element_type=jnp.float32)
    m = _silu(r[:, :64])
    scol = r[:, 64:65]
    m = _silu(jnp.dot(m, w2_ref[:], preferred_element_type=jnp.float32))
    m = jnp.dot(m, w3_ref[:], preferred_element_type=jnp.float32)
    m = m * INV_SQRT_AVG
    msgs = msgs_ref[:]
    wd = wd_ref[:]
    y_ref[:] = (
        jnp.dot(msgs * m[:, :D], wd[:D], preferred_element_type=jnp.float32)
        + jnp.dot(msgs * (m[:, D:] * scol), wd[D:],
                  preferred_element_type=jnp.float32))


def _edge_tc(vectors_t, msgs, w_sh2d, Q1, W2, W3, W_down):
    grid = (E // EB,)
    return pl.pallas_call(
        _edge_tc_body,
        grid=grid,
        in_specs=[
            pl.BlockSpec((3, EB), lambda i: (0, i)),
            pl.BlockSpec((EB, D), lambda i: (i, 0)),
            pl.BlockSpec((1, 15), lambda i: (0, 0)),
            pl.BlockSpec((9, 128), lambda i: (0, 0)),
            pl.BlockSpec((64, 64), lambda i: (0, 0)),
            pl.BlockSpec((64, 2 * D), lambda i: (0, 0)),
            pl.BlockSpec((2 * D, D), lambda i: (0, 0)),
        ],
        out_specs=pl.BlockSpec((EB, D), lambda i: (i, 0)),
        out_shape=jax.ShapeDtypeStruct((E, D), jnp.float32),
    )(vectors_t, msgs, w_sh2d, Q1, W2, W3, W_down)


@functools.cache
def _make_sc_scatter():
    mesh = plsc.VectorSubcoreMesh(core_axis_name="c", subcore_axis_name="s",
                                  num_cores=NCORE, num_subcores=NSUB)
    return functools.partial(
        pl.kernel,
        mesh=mesh,
        out_type=jax.ShapeDtypeStruct((NCORE, NPAD, D), jnp.float32),
        scratch_types=[
            pltpu.VMEM((SCB,), jnp.int32),
            pltpu.VMEM((SCB, D), jnp.float32),
            pltpu.VMEM_SHARED((NPAD, D), jnp.float32),
        ],
    )(_sc_scatter_body)


def _sc_scatter_body(y_hbm, rcv_hbm, out_hbm, ridx, rows, aggsh):
    c = lax.axis_index("c")
    t = lax.axis_index("s")

    def _zero_row(r, carry):
        for cc in range(D // 16):
            rows[r, pl.ds(cc * 16, 16)] = jnp.zeros((16,), jnp.float32)
        return carry

    lax.fori_loop(0, SCB, _zero_row, 0)
    base = t * ROWS_PER_TEC
    for j in range(ROWS_PER_TEC // SCB):
        pltpu.sync_copy(rows, aggsh.at[pl.ds(base + j * SCB, SCB)])
    plsc.subcore_barrier()

    half = NBLK // NCORE

    def _block(i, carry):
        j = t + i * NSUB

        @pl.when(j < half)
        def _():
            off = (c * half + j) * SCB
            pltpu.sync_copy(rcv_hbm.at[pl.ds(off, SCB)], ridx)
            pltpu.sync_copy(y_hbm.at[pl.ds(off, SCB)], rows)
            pltpu.sync_copy(rows, aggsh.at[ridx], add=True)

        return carry

    lax.fori_loop(0, (half + NSUB - 1) // NSUB, _block, 0)
    plsc.subcore_barrier()
    pltpu.sync_copy(aggsh.at[pl.ds(base, ROWS_PER_TEC)],
                    out_hbm.at[c, pl.ds(base, ROWS_PER_TEC)])


def _final_body(agg_ref, sc_ref, out_ref):
    out_ref[:] = _silu(agg_ref[0, :N] + agg_ref[1, :N]) + sc_ref[:]


def _final(agg, sc):
    return pl.pallas_call(
        _final_body,
        out_shape=jax.ShapeDtypeStruct((N, D), jnp.float32),
    )(agg, sc)


def kernel(vectors, node_feats, node_specie, senders, receivers,
           W_up, w_sh, W1, W2, W3, W_down, W_skip):
    h, sc = _node_prep(node_feats,
                       node_specie.reshape(N, 1).astype(jnp.int32),
                       W_up, W_skip)
    msgs = _make_sc_gather()(h, senders.astype(jnp.int32))
    Q1 = jnp.zeros((9, 128), jnp.float32)
    Q1 = Q1.at[:8, :64].set(W1).at[8, 64].set(1.0)
    y = _edge_tc(vectors.T, msgs, w_sh.reshape(1, 15), Q1, W2, W3, W_down)
    agg = _make_sc_scatter()(y, receivers.astype(jnp.int32))
    return _final(agg, sc)


# speedup vs baseline: 5.0475x; 1.3114x over previous
import functools
import math

import jax
import jax.numpy as jnp
from jax import lax
from jax.experimental import pallas as pl
from jax.experimental.pallas import tpu as pltpu
from jax.experimental.pallas import tpu_sc as plsc

N = 10000
E = 320000
D = 128
NC = 4
ECH = E // NC
EB = 3200
SCB = 128
CBLK = ECH // SCB
NCORE = 2
NSUB = 16
NWORK = NCORE * NSUB
NPAD = 10240
ROWS_PER_TEC = NPAD // NSUB
INV_SQRT_AVG = 1.0 / math.sqrt(32.0)


def _silu(x):
    return x * jax.nn.sigmoid(x)


def _node_prep_body(nf_ref, sp_ref, wup_ref, wskip_ref, h_ref, sc_ref):
    nf = nf_ref[:]
    h_ref[:] = jnp.dot(nf, wup_ref[:], preferred_element_type=jnp.float32)
    sp = sp_ref[:]
    acc = jnp.zeros_like(nf)
    for k in range(5):
        masked = jnp.where(sp == k, nf, 0.0)
        acc = acc + jnp.dot(masked, wskip_ref[k], preferred_element_type=jnp.float32)
    sc_ref[:] = acc


def _node_prep(node_feats, node_specie2d, W_up, W_skip):
    return pl.pallas_call(
        _node_prep_body,
        out_shape=[
            jax.ShapeDtypeStruct((N, D), jnp.float32),
            jax.ShapeDtypeStruct((N, D), jnp.float32),
        ],
    )(node_feats, node_specie2d, W_up, W_skip)


@functools.cache
def _make_sc_gather():
    mesh = plsc.VectorSubcoreMesh(core_axis_name="c", subcore_axis_name="s",
                                  num_cores=NCORE, num_subcores=NSUB)
    return functools.partial(
        pl.kernel,
        mesh=mesh,
        out_type=jax.ShapeDtypeStruct((ECH, D), jnp.float32),
        scratch_types=[
            pltpu.VMEM((SCB,), jnp.int32),
            pltpu.VMEM((SCB, D), jnp.float32),
            pltpu.SemaphoreType.DMA,
        ],
    )(_sc_gather_body)


def _sc_gather_body(h_hbm, snd_hbm, out_hbm, sidx, rows, sem):
    c = lax.axis_index("c")
    t = lax.axis_index("s")
    w = t * NCORE + c

    def _block(i, carry):
        b = w + i * NWORK

        @pl.when(b < CBLK)
        def _():
            off = b * SCB
            pltpu.sync_copy(snd_hbm.at[pl.ds(off, SCB)], sidx)
            pltpu.async_copy(h_hbm.at[sidx], rows, sem).wait()
            pltpu.sync_copy(rows, out_hbm.at[pl.ds(off, SCB)])

        return carry

    lax.fori_loop(0, (CBLK + NWORK - 1) // NWORK, _block, 0)


_SH_COEF = [
    math.sqrt(3.0), math.sqrt(3.0), math.sqrt(3.0),
    math.sqrt(15.0), math.sqrt(15.0), math.sqrt(5.0) / 2.0,
    math.sqrt(15.0), math.sqrt(15.0) / 2.0,
    math.sqrt(35.0 / 8.0), math.sqrt(105.0), math.sqrt(21.0 / 8.0),
    math.sqrt(7.0) / 2.0, math.sqrt(21.0 / 8.0), math.sqrt(105.0) / 2.0,
    math.sqrt(35.0 / 8.0),
]


def _edge_tc_body(vt_ref, msgs_ref, wsh_ref, q1_ref, w2_ref, w3_ref, wd_ref,
                  y_ref):
    x = vt_ref[0:1]
    y = vt_ref[1:2]
    z = vt_ref[2:3]
    x2 = x * x + y * y + z * z
    l = jnp.sqrt(jnp.where(x2 == 0.0, 1.0, x2))
    invl = 1.0 / l
    ns = (lax.broadcasted_iota(jnp.int32, (8, 1), 0) + 1).astype(jnp.float32)
    l2 = l * l
    l5 = l2 * l2 * l
    env = 1.0 - 21.0 * l5 + 35.0 * l5 * l - 15.0 * l5 * l2
    cutoff = jnp.where(l < 1.0, env, 0.0)
    radT = (jnp.sqrt(2.0) * jnp.sin(ns * (jnp.pi * l))) * (cutoff * invl)
    ux, uy, uz = x * invl, y * invl, z * invl
    comps = [
        ux, uy, uz,
        ux * uy, uy * uz, 3.0 * uz * uz - 1.0, ux * uz, ux * ux - uy * uy,
        uy * (3.0 * ux * ux - uy * uy), ux * uy * uz,
        uy * (5.0 * uz * uz - 1.0), uz * (5.0 * uz * uz - 3.0),
        ux * (5.0 * uz * uz - 1.0), uz * (ux * ux - uy * uy),
        ux * (ux * ux - 3.0 * uy * uy),
    ]
    s = jnp.zeros_like(x)
    for k in range(15):
        s = s + (wsh_ref[0:1, k:k + 1] * _SH_COEF[k]) * comps[k]
    p = jnp.concatenate([radT, s], axis=0)
    r = lax.dot_general(p, q1_ref[:], (((0,), (0,)), ((), ())),
                        preferred_element_type=jnp.float32)
    m = _silu(r[:, :64])
    scol = r[:, 64:65]
    m = _silu(jnp.dot(m, w2_ref[:], preferred_element_type=jnp.float32))
    m = jnp.dot(m, w3_ref[:], preferred_element_type=jnp.float32)
    m = m * INV_SQRT_AVG
    msgs = msgs_ref[:]
    wd = wd_ref[:]
    y_ref[:] = (
        jnp.dot(msgs * m[:, :D], wd[:D], preferred_element_type=jnp.float32)
        + jnp.dot(msgs * (m[:, D:] * scol), wd[D:],
                  preferred_element_type=jnp.float32))


def _edge_tc(vectors_t, msgs, w_sh2d, Q1, W2, W3, W_down):
    grid = (ECH // EB,)
    return pl.pallas_call(
        _edge_tc_body,
        grid=grid,
        in_specs=[
            pl.BlockSpec((3, EB), lambda i: (0, i)),
            pl.BlockSpec((EB, D), lambda i: (i, 0)),
            pl.BlockSpec((1, 15), lambda i: (0, 0)),
            pl.BlockSpec((9, 128), lambda i: (0, 0)),
            pl.BlockSpec((64, 64), lambda i: (0, 0)),
            pl.BlockSpec((64, 2 * D), lambda i: (0, 0)),
            pl.BlockSpec((2 * D, D), lambda i: (0, 0)),
        ],
        out_specs=pl.BlockSpec((EB, D), lambda i: (i, 0)),
        out_shape=jax.ShapeDtypeStruct((ECH, D), jnp.float32),
    )(vectors_t, msgs, w_sh2d, Q1, W2, W3, W_down)


@functools.cache
def _make_sc_scatter():
    mesh = plsc.VectorSubcoreMesh(core_axis_name="c", subcore_axis_name="s",
                                  num_cores=NCORE, num_subcores=NSUB)
    return functools.partial(
        pl.kernel,
        mesh=mesh,
        out_type=jax.ShapeDtypeStruct((NCORE, NPAD, D), jnp.float32),
        scratch_types=[
            pltpu.VMEM((SCB,), jnp.int32),
            pltpu.VMEM((SCB, D), jnp.float32),
            pltpu.VMEM_SHARED((NPAD, D), jnp.float32),
        ],
    )(_sc_scatter_body)


def _sc_scatter_body(ya_hbm, yb_hbm, rcva_hbm, rcvb_hbm, out_hbm,
                     ridx, rows, aggsh):
    c = lax.axis_index("c")
    t = lax.axis_index("s")

    def _zero_row(r, carry):
        for cc in range(D // 16):
            rows[r, pl.ds(cc * 16, 16)] = jnp.zeros((16,), jnp.float32)
        return carry

    lax.fori_loop(0, SCB, _zero_row, 0)
    base = t * ROWS_PER_TEC
    for j in range(ROWS_PER_TEC // SCB):
        pltpu.sync_copy(rows, aggsh.at[pl.ds(base + j * SCB, SCB)])
    plsc.subcore_barrier()

    def _block(i, carry):
        j = t + i * NSUB

        @pl.when(j < CBLK)
        def _():
            off = j * SCB

            @pl.when(c == 0)
            def _():
                pltpu.sync_copy(rcva_hbm.at[pl.ds(off, SCB)], ridx)
                pltpu.sync_copy(ya_hbm.at[pl.ds(off, SCB)], rows)

            @pl.when(c == 1)
            def _():
                pltpu.sync_copy(rcvb_hbm.at[pl.ds(off, SCB)], ridx)
                pltpu.sync_copy(yb_hbm.at[pl.ds(off, SCB)], rows)

            pltpu.sync_copy(rows, aggsh.at[ridx], add=True)

        return carry

    lax.fori_loop(0, (CBLK + NSUB - 1) // NSUB, _block, 0)
    plsc.subcore_barrier()
    pltpu.sync_copy(aggsh.at[pl.ds(base, ROWS_PER_TEC)],
                    out_hbm.at[c, pl.ds(base, ROWS_PER_TEC)])


def _final_body(agg0_ref, agg1_ref, sc_ref, out_ref):
    tot = (agg0_ref[0, :N] + agg0_ref[1, :N]
           + agg1_ref[0, :N] + agg1_ref[1, :N])
    out_ref[:] = _silu(tot) + sc_ref[:]


def _final(agg0, agg1, sc):
    return pl.pallas_call(
        _final_body,
        out_shape=jax.ShapeDtypeStruct((N, D), jnp.float32),
    )(agg0, agg1, sc)


def kernel(vectors, node_feats, node_specie, senders, receivers,
           W_up, w_sh, W1, W2, W3, W_down, W_skip):
    h, sc = _node_prep(node_feats,
                       node_specie.reshape(N, 1).astype(jnp.int32),
                       W_up, W_skip)
    snd = senders.astype(jnp.int32)
    rcv = receivers.astype(jnp.int32)
    vt = vectors.T
    Q1 = jnp.zeros((9, 128), jnp.float32)
    Q1 = Q1.at[:8, :64].set(W1).at[8, 64].set(1.0)
    w_sh2d = w_sh.reshape(1, 15)

    gather = _make_sc_gather()
    ys = []
    for ci in range(NC):
        lo = ci * ECH
        msgs = gather(h, lax.slice(snd, (lo,), (lo + ECH,)))
        ys.append(_edge_tc(lax.slice(vt, (0, lo), (3, lo + ECH)),
                           msgs, w_sh2d, Q1, W2, W3, W_down))

    scatter = _make_sc_scatter()
    rchunk = [lax.slice(rcv, (ci * ECH,), ((ci + 1) * ECH,))
              for ci in range(NC)]
    agg0 = scatter(ys[0], ys[1], rchunk[0], rchunk[1])
    agg1 = scatter(ys[2], ys[3], rchunk[2], rchunk[3])
    return _final(agg0, agg1, sc)
